# baseline (speedup 1.0000x reference)
import functools

import jax
import jax.numpy as jnp
from jax import lax
from jax.experimental import pallas as pl
from jax.experimental.pallas import tpu as pltpu
from jax.experimental.pallas import tpu_sc as plsc

_L0 = 1000
_GROUP = 18432
_BLOCK = 2304
_MAXN = 512
_NGROUP = 8
_LANES = 16
_NROWS = 2 * _NGROUP


def _row_params(wid, g, t):
    lr = 2 * wid + t
    m = lax.rem(lr, 8)
    blk = lax.div(lr, 8)
    n = 64 * (m + 1)
    in_off = _L0 + g * _GROUP + blk * _BLOCK + 32 * m * (m + 1)
    return lr, n, in_off


def _body(in_hbm, *refs):
    out0 = refs[0]
    outs = refs[1:1 + _NGROUP]
    rows_v = refs[1 + _NGROUP]
    l0_v = refs[2 + _NGROUP]
    sem_in = refs[3 + _NGROUP]
    sem_out = refs[4 + _NGROUP]
    sem_l0 = refs[5 + _NGROUP]

    wid = lax.axis_index("s") * 2 + lax.axis_index("c")

    neg_inf = jnp.full((_LANES,), -jnp.inf, dtype=jnp.float32)
    lane = lax.iota(jnp.int32, _LANES)

    gathers = []
    for g in range(_NGROUP):
        for t in range(2):
            _, _, in_off = _row_params(wid, g, t)
            gathers.append(pltpu.async_copy(
                in_hbm.at[pl.ds(in_off, _MAXN)],
                rows_v.at[pl.ds((2 * g + t) * _MAXN, _MAXN)], sem_in))

    @pl.when(wid == 0)
    def _():
        pltpu.async_copy(in_hbm.at[pl.ds(0, _L0)], l0_v, sem_l0).wait()
        pltpu.async_copy(l0_v, out0, sem_l0).wait()

    for cp in gathers:
        cp.wait()

    for g in range(_NGROUP):
        for t in range(2):
            _, n, _ = _row_params(wid, g, t)
            base = (2 * g + t) * _MAXN
            for c in range(4, _MAXN // _LANES):
                v = rows_v[pl.ds(base + c * _LANES, _LANES)]
                keep = (lane + (c * _LANES)) < n
                rows_v[pl.ds(base + c * _LANES, _LANES)] = (
                    jnp.where(keep, v, neg_inf))

    scatters = []
    for g in range(_NGROUP):
        lr0, _, _ = _row_params(wid, g, 0)
        scatters.append(pltpu.async_copy(
            rows_v.at[pl.ds(2 * g * _MAXN, 2 * _MAXN)],
            outs[g].at[pl.ds(lr0 * _MAXN, 2 * _MAXN)], sem_out))
    for cp in scatters:
        cp.wait()


_OUT_TYPE = (
    (jax.ShapeDtypeStruct((_L0,), jnp.float32),)
    + tuple(jax.ShapeDtypeStruct((64 * _MAXN,), jnp.float32)
            for _ in range(_NGROUP))
)

_sc_interpret = functools.partial(
    pl.kernel,
    mesh=plsc.VectorSubcoreMesh(core_axis_name="c", subcore_axis_name="s"),
    out_type=_OUT_TYPE,
    scratch_types=[
        pltpu.VMEM((_NROWS * _MAXN,), jnp.float32),
        pltpu.VMEM((_L0,), jnp.float32),
        pltpu.SemaphoreType.DMA,
        pltpu.SemaphoreType.DMA,
        pltpu.SemaphoreType.DMA,
    ],
)(_body)


def kernel(logits):
    flat = _sc_interpret(logits)
    return ((flat[0].reshape(1, _L0),)
            + tuple(f.reshape(64, _MAXN) for f in flat[1:]))


# speedup vs baseline: 1.2920x; 1.2920x over previous
import functools

import jax
import jax.numpy as jnp
from jax import lax
from jax.experimental import pallas as pl
from jax.experimental.pallas import tpu as pltpu
from jax.experimental.pallas import tpu_sc as plsc

_L0 = 1000
_GROUP = 18432
_BLOCK = 2304
_MAXN = 512
_NGROUP = 8
_LANES = 16
_NROWS = 2 * _NGROUP


def _row_params(wid, g, t):
    lr = 2 * wid + t
    m = lax.rem(lr, 8)
    blk = lax.div(lr, 8)
    n = 64 * (m + 1)
    in_off = _L0 + g * _GROUP + blk * _BLOCK + 32 * m * (m + 1)
    return lr, n, in_off


def _body(in_hbm, *refs):
    out0 = refs[0]
    outs = refs[1:1 + _NGROUP]
    rows_v = refs[1 + _NGROUP]
    l0_v = refs[2 + _NGROUP]
    sem_in = refs[3 + _NGROUP]
    sem_out = refs[4 + _NGROUP]
    sem_l0 = refs[5 + _NGROUP]

    wid = lax.axis_index("s") * 2 + lax.axis_index("c")

    neg_inf = jnp.full((_LANES,), -jnp.inf, dtype=jnp.float32)
    lane = lax.iota(jnp.int32, _LANES)

    gathers = []
    for g in range(_NGROUP):
        for t in range(2):
            _, _, in_off = _row_params(wid, g, t)
            gathers.append(pltpu.async_copy(
                in_hbm.at[pl.ds(in_off, _MAXN)],
                rows_v.at[pl.ds((2 * g + t) * _MAXN, _MAXN)], sem_in))

    @pl.when(wid == 0)
    def _():
        pltpu.async_copy(in_hbm.at[pl.ds(0, _L0)], l0_v, sem_l0).wait()
        pltpu.async_copy(l0_v, out0.at[0], sem_l0).wait()

    for cp in gathers:
        cp.wait()

    for g in range(_NGROUP):
        for t in range(2):
            _, n, _ = _row_params(wid, g, t)
            base = (2 * g + t) * _MAXN
            for c in range(4, _MAXN // _LANES):
                v = rows_v[pl.ds(base + c * _LANES, _LANES)]
                keep = (lane + (c * _LANES)) < n
                rows_v[pl.ds(base + c * _LANES, _LANES)] = (
                    jnp.where(keep, v, neg_inf))

    scatters = []
    for g in range(_NGROUP):
        for t in range(2):
            lr, _, _ = _row_params(wid, g, t)
            scatters.append(pltpu.async_copy(
                rows_v.at[pl.ds((2 * g + t) * _MAXN, _MAXN)],
                outs[g].at[lr], sem_out))
    for cp in scatters:
        cp.wait()


_OUT_TYPE = (
    (jax.ShapeDtypeStruct((1, _L0), jnp.float32),)
    + tuple(jax.ShapeDtypeStruct((64, _MAXN), jnp.float32)
            for _ in range(_NGROUP))
)

_sc_interpret = functools.partial(
    pl.kernel,
    mesh=plsc.VectorSubcoreMesh(core_axis_name="c", subcore_axis_name="s"),
    out_type=_OUT_TYPE,
    scratch_types=[
        pltpu.VMEM((_NROWS * _MAXN,), jnp.float32),
        pltpu.VMEM((_L0,), jnp.float32),
        pltpu.SemaphoreType.DMA,
        pltpu.SemaphoreType.DMA,
        pltpu.SemaphoreType.DMA,
    ],
)(_body)


def kernel(logits):
    return _sc_interpret(logits)


# speedup vs baseline: 1.4279x; 1.1052x over previous
import functools

import jax
import jax.numpy as jnp
from jax import lax
from jax.experimental import pallas as pl
from jax.experimental.pallas import tpu as pltpu
from jax.experimental.pallas import tpu_sc as plsc

_L0 = 1000
_GROUP = 18432
_BLOCK = 2304
_MAXN = 512
_NGROUP = 8
_LANES = 16
_NROWS = 2 * _NGROUP


def _body(in_hbm, *refs):
    out0 = refs[0]
    outs = refs[1:1 + _NGROUP]
    rows_v = refs[1 + _NGROUP]
    l0_v = refs[2 + _NGROUP]
    sem_in = refs[3 + _NGROUP]
    sem_out = refs[4 + _NGROUP]
    sem_l0 = refs[5 + _NGROUP]

    wid = lax.axis_index("s") * 2 + lax.axis_index("c")

    neg_inf = jnp.full((_LANES,), -jnp.inf, dtype=jnp.float32)

    def _fire(i, _):
        t = lax.rem(i, 2)
        g = lax.div(i, 2)
        lr = 2 * wid + t
        m = lax.rem(lr, 8)
        blk = lax.div(lr, 8)
        in_off = _L0 + g * _GROUP + blk * _BLOCK + 32 * m * (m + 1)
        pltpu.async_copy(in_hbm.at[pl.ds(in_off, _MAXN)],
                         rows_v.at[pl.ds(i * _MAXN, _MAXN)], sem_in)
        return 0

    lax.fori_loop(0, _NROWS, _fire, 0)

    @pl.when(wid == 0)
    def _():
        pltpu.async_copy(in_hbm.at[pl.ds(0, _L0)], l0_v, sem_l0).wait()
        pltpu.async_copy(l0_v, out0.at[0], sem_l0).wait()

    pltpu.make_async_copy(in_hbm.at[pl.ds(0, _NROWS * _MAXN)],
                          rows_v, sem_in).wait()

    def _pad_row(i, _):
        m = lax.rem(2 * wid + lax.rem(i, 2), 8)
        base = i * _MAXN

        def _pad(c, _):
            rows_v[pl.ds(base + c * _LANES, _LANES)] = neg_inf
            return 0

        lax.fori_loop(4 * (m + 1), _MAXN // _LANES, _pad, 0)
        return 0

    lax.fori_loop(0, _NROWS, _pad_row, 0)

    for g in range(_NGROUP):
        for t in range(2):
            pltpu.async_copy(rows_v.at[pl.ds((2 * g + t) * _MAXN, _MAXN)],
                             outs[g].at[2 * wid + t], sem_out)
    pltpu.make_async_copy(in_hbm.at[pl.ds(0, _NROWS * _MAXN)],
                          rows_v, sem_out).wait()


_OUT_TYPE = (
    (jax.ShapeDtypeStruct((1, _L0), jnp.float32),)
    + tuple(jax.ShapeDtypeStruct((64, _MAXN), jnp.float32)
            for _ in range(_NGROUP))
)

_sc_interpret = functools.partial(
    pl.kernel,
    mesh=plsc.VectorSubcoreMesh(core_axis_name="c", subcore_axis_name="s"),
    out_type=_OUT_TYPE,
    scratch_types=[
        pltpu.VMEM((_NROWS * _MAXN,), jnp.float32),
        pltpu.VMEM((_L0,), jnp.float32),
        pltpu.SemaphoreType.DMA,
        pltpu.SemaphoreType.DMA,
        pltpu.SemaphoreType.DMA,
    ],
)(_body)


def kernel(logits):
    return _sc_interpret(logits)


# speedup vs baseline: 1.5193x; 1.0640x over previous
import functools

import jax
import jax.numpy as jnp
from jax import lax
from jax.experimental import pallas as pl
from jax.experimental.pallas import tpu as pltpu
from jax.experimental.pallas import tpu_sc as plsc

_L0 = 1000
_GROUP = 18432
_BLOCK = 2304
_MAXN = 512
_NGROUP = 8
_LANES = 16
_NROWS = 2 * _NGROUP


def _body(in_hbm, *refs):
    out0 = refs[0]
    outs = refs[1:1 + _NGROUP]
    rows_v = refs[1 + _NGROUP]
    l0_v = refs[2 + _NGROUP]
    sem_in = refs[3 + _NGROUP]
    sem_out = refs[4 + _NGROUP]
    sem_l0 = refs[5 + _NGROUP]

    wid = lax.axis_index("s") * 2 + lax.axis_index("c")

    neg_inf = jnp.full((_LANES,), -jnp.inf, dtype=jnp.float32)

    def _fire(i, _):
        t = lax.rem(i, 2)
        g = lax.div(i, 2)
        lr = wid + t * (63 - 2 * wid)
        m = lax.rem(lr, 8)
        blk = lax.div(lr, 8)
        in_off = _L0 + g * _GROUP + blk * _BLOCK + 32 * m * (m + 1)
        pltpu.async_copy(in_hbm.at[pl.ds(in_off, _MAXN)],
                         rows_v.at[pl.ds(i * _MAXN, _MAXN)], sem_in)
        return 0

    lax.fori_loop(0, _NROWS, _fire, 0)

    @pl.when(wid == 0)
    def _():
        pltpu.async_copy(in_hbm.at[pl.ds(0, _L0)], l0_v, sem_l0)

    pltpu.make_async_copy(in_hbm.at[pl.ds(0, _NROWS * _MAXN)],
                          rows_v, sem_in).wait()

    @pl.when(wid == 0)
    def _():
        pltpu.make_async_copy(in_hbm.at[pl.ds(0, _L0)], l0_v, sem_l0).wait()
        pltpu.async_copy(l0_v, out0.at[0], sem_l0)

    for g in range(_NGROUP):
        for t in range(2):
            lr = wid + t * (63 - 2 * wid)
            m = lax.rem(lr, 8)
            base = (2 * g + t) * _MAXN

            def _pad64(c, _, base=base):
                for k in range(4):
                    rows_v[pl.ds(base + c * 64 + k * _LANES,
                                 _LANES)] = neg_inf
                return 0

            lax.fori_loop(m + 1, 8, _pad64, 0)
            pltpu.async_copy(rows_v.at[pl.ds(base, _MAXN)],
                             outs[g].at[lr], sem_out)

    pltpu.make_async_copy(in_hbm.at[pl.ds(0, _NROWS * _MAXN)],
                          rows_v, sem_out).wait()

    @pl.when(wid == 0)
    def _():
        pltpu.make_async_copy(in_hbm.at[pl.ds(0, _L0)], l0_v, sem_l0).wait()


_OUT_TYPE = (
    (jax.ShapeDtypeStruct((1, _L0), jnp.float32),)
    + tuple(jax.ShapeDtypeStruct((64, _MAXN), jnp.float32)
            for _ in range(_NGROUP))
)

_sc_interpret = functools.partial(
    pl.kernel,
    mesh=plsc.VectorSubcoreMesh(core_axis_name="c", subcore_axis_name="s"),
    out_type=_OUT_TYPE,
    scratch_types=[
        pltpu.VMEM((_NROWS * _MAXN,), jnp.float32),
        pltpu.VMEM((_L0,), jnp.float32),
        pltpu.SemaphoreType.DMA,
        pltpu.SemaphoreType.DMA,
        pltpu.SemaphoreType.DMA,
    ],
)(_body)


def kernel(logits):
    return _sc_interpret(logits)
